# Initial kernel scaffold; baseline (speedup 1.0000x reference)
import jax
import jax.numpy as jnp
from jax.experimental import pallas as pl


def kernel(x, embeddings):
    raise NotImplementedError("write your pallas kernel here")


# speedup vs baseline: 3.0303x; 3.0303x over previous
import jax
import jax.numpy as jnp
from jax.experimental import pallas as pl
from jax.experimental.pallas import tpu as pltpu

EMB_D = 64
EMB_K = 1024
VQ_BETA = 0.25


def _vq_body(x_ref, emb_ref, out_ref, loss_ref):
    b = pl.program_id(0)
    x_b = x_ref[0]
    emb = emb_ref[...]

    e_sq = jnp.sum(emb * emb, axis=1, keepdims=True)
    neg2xe = jax.lax.dot_general(
        emb, x_b, (((1,), (0,)), ((), ())),
        preferred_element_type=jnp.float32)
    dist = e_sq - 2.0 * neg2xe

    mn = jnp.min(dist, axis=0, keepdims=True)
    k_iota = jax.lax.broadcasted_iota(jnp.int32, dist.shape, 0)
    cand = jnp.where(dist == mn, k_iota, EMB_K)
    idx = jnp.min(cand, axis=0, keepdims=True)
    onehot = (k_iota == idx).astype(jnp.float32)

    q = jax.lax.dot_general(
        emb, onehot, (((0,), (0,)), ((), ())),
        preferred_element_type=jnp.float32)

    out_ref[0] = q
    diff = q - x_b
    part = jnp.sum(diff * diff)

    @pl.when(b == 0)
    def _init():
        loss_ref[0, 0] = 0.0

    loss_ref[0, 0] += part


def kernel(x, embeddings):
    B = x.shape[0]
    T = x.shape[-1]
    xs = x.reshape(B, EMB_D, T)

    q, loss_sum = pl.pallas_call(
        _vq_body,
        grid=(B,),
        in_specs=[
            pl.BlockSpec((1, EMB_D, T), lambda b: (b, 0, 0)),
            pl.BlockSpec((EMB_K, EMB_D), lambda b: (0, 0)),
        ],
        out_specs=[
            pl.BlockSpec((1, EMB_D, T), lambda b: (b, 0, 0)),
            pl.BlockSpec(
                block_shape=(1, 1),
                index_map=lambda b: (0, 0),
                memory_space=pltpu.SMEM,
            ),
        ],
        out_shape=[
            jax.ShapeDtypeStruct((B, EMB_D, T), jnp.float32),
            jax.ShapeDtypeStruct((1, 1), jnp.float32),
        ],
    )(xs, embeddings)

    loss = loss_sum[0, 0] * ((1.0 + VQ_BETA) / (B * T * EMB_D))
    return (q, loss)
